# Initial kernel scaffold; baseline (speedup 1.0000x reference)
import jax
import jax.numpy as jnp
from jax.experimental import pallas as pl


def kernel(x, edge_index, W1, b1, W2, b2, gru1_Wih, gru1_Whh, gru1_bih, gru1_bhh, wt1_W, wt1_b, gcn1_b, gru2_Wih, gru2_Whh, gru2_bih, gru2_bhh, wt2_W, wt2_b, gcn2_b, Wp, bp, Wa, ba):
    raise NotImplementedError("write your pallas kernel here")


# speedup vs baseline: 27.9922x; 27.9922x over previous
import functools

import jax
import jax.numpy as jnp
from jax import lax
from jax.experimental import pallas as pl
from jax.experimental.pallas import tpu as pltpu
from jax.experimental.pallas import tpu_sc as plsc

N = 10000
E = 320000
D = 128
H = 128

NC = 2
NS = 16
NW = NC * NS
CHUNK = 128
CPT = 80
E_PAD = NW * CPT * CHUNK
N_PAD = 10240
RPT = N_PAD // NS
BLK = 512
GRID = N_PAD // BLK

_mesh = plsc.VectorSubcoreMesh(
    core_axis_name="c", subcore_axis_name="s", num_cores=NC, num_subcores=NS)


@functools.partial(
    pl.kernel,
    out_type=jax.ShapeDtypeStruct((NC, N_PAD), jnp.float32),
    mesh=_mesh,
    scratch_types=[
        pltpu.VMEM_SHARED((N_PAD,), jnp.float32),
        pltpu.VMEM((CPT, CHUNK), jnp.int32),
        pltpu.VMEM((CHUNK,), jnp.float32),
    ],
)
def _deg_kernel(dst_hbm, zeros1_hbm, deg_hbm, acc, idx, ones):
    c = lax.axis_index("c")
    s = lax.axis_index("s")
    wid = c * NS + s
    pltpu.sync_copy(zeros1_hbm.at[pl.ds(s * RPT, RPT)],
                    acc.at[pl.ds(s * RPT, RPT)])
    pltpu.sync_copy(dst_hbm.at[pl.ds(wid * CPT, CPT)], idx)
    for j in range(CHUNK // 16):
        ones[pl.ds(j * 16, 16)] = jnp.ones((16,), jnp.float32)
    plsc.subcore_barrier()

    def body(k, carry):
        pltpu.sync_copy(ones, acc.at[idx.at[k]], add=True)
        return carry

    lax.fori_loop(0, CPT, body, 0)
    plsc.subcore_barrier()
    pltpu.sync_copy(acc.at[pl.ds(s * RPT, RPT)],
                    deg_hbm.at[c, pl.ds(s * RPT, RPT)])


GRP = 8
NGRP = CPT // GRP


@functools.partial(
    pl.kernel,
    out_type=jax.ShapeDtypeStruct((NC, N_PAD, H), jnp.float32),
    mesh=_mesh,
    scratch_types=[
        pltpu.VMEM_SHARED((N_PAD, H), jnp.float32),
        pltpu.VMEM((2, GRP, CHUNK), jnp.int32),
        pltpu.VMEM((2, GRP, CHUNK), jnp.int32),
        pltpu.VMEM((2, CHUNK, H), jnp.float32),
        pltpu.SemaphoreType.DMA,
        pltpu.SemaphoreType.DMA,
    ],
)
def _agg_kernel(y_hbm, src_hbm, dst_hbm, zrows_hbm, out_hbm,
                acc, idxs, idxd, rows, gsem, isem):
    c = lax.axis_index("c")
    s = lax.axis_index("s")
    wid = c * NS + s
    base = wid * CPT
    pltpu.sync_copy(zrows_hbm, acc.at[pl.ds(s * RPT, RPT)])
    pltpu.sync_copy(src_hbm.at[pl.ds(base, GRP)], idxs.at[0])
    pltpu.sync_copy(dst_hbm.at[pl.ds(base, GRP)], idxd.at[0])
    plsc.subcore_barrier()

    pltpu.async_copy(y_hbm.at[idxs.at[0].at[0]], rows.at[0], gsem)

    def body(k, carry):
        g = k // GRP
        j = k - g * GRP
        gb = lax.rem(g, 2)
        kn = k + 1
        gn = kn // GRP
        jn = kn - gn * GRP
        gbn = lax.rem(gn, 2)

        @pl.when(jnp.logical_and(j == 0, g + 1 < NGRP))
        def _():
            off = base + (g + 1) * GRP
            pltpu.async_copy(src_hbm.at[pl.ds(off, GRP)], idxs.at[1 - gb],
                             isem)
            pltpu.async_copy(dst_hbm.at[pl.ds(off, GRP)], idxd.at[1 - gb],
                             isem)

        @pl.when(jnp.logical_and(j == GRP - 1, g + 1 < NGRP))
        def _():
            off = base + (g + 1) * GRP
            pltpu.make_async_copy(src_hbm.at[pl.ds(off, GRP)],
                                  idxs.at[1 - gb], isem).wait()
            pltpu.make_async_copy(dst_hbm.at[pl.ds(off, GRP)],
                                  idxd.at[1 - gb], isem).wait()

        @pl.when(kn < CPT)
        def _():
            pltpu.async_copy(y_hbm.at[idxs.at[gbn].at[jn]],
                             rows.at[lax.rem(kn, 2)], gsem)

        b = lax.rem(k, 2)
        pltpu.make_async_copy(y_hbm.at[idxs.at[gb].at[j]],
                              rows.at[b], gsem).wait()
        pltpu.sync_copy(rows.at[b], acc.at[idxd.at[gb].at[j]], add=True)
        return carry

    lax.fori_loop(0, CPT, body, 0)
    plsc.subcore_barrier()
    pltpu.sync_copy(acc.at[pl.ds(s * RPT, RPT)],
                    out_hbm.at[c, pl.ds(s * RPT, RPT)])


def _lrelu(v):
    return jnp.where(v > 0, v, 0.01 * v)


def _wgen_body(bihr, bihz, bihn, bhhr, bhhz, bhhn, wtT, wtb, w_out):
    z = jax.nn.sigmoid(bihz[...] + bhhz[...])
    r = jax.nn.sigmoid(bihr[...] + bhhr[...])
    n = jnp.tanh(bihn[...] + r * bhhn[...])
    mem = (1.0 - z) * n
    w_out[...] = jnp.dot(mem, wtT[...],
                         preferred_element_type=jnp.float32) + wtb[...]


def _stage_a_body(x_ref, w1t, b1, w2t, b2, wg1t, degT, y1_ref, dinv_ref):
    deg = jnp.sum(degT[...], axis=1, keepdims=True) + 1.0
    dv = lax.rsqrt(deg)
    h = _lrelu(jnp.dot(x_ref[...], w1t[...],
                       preferred_element_type=jnp.float32) + b1[...])
    h = _lrelu(jnp.dot(h, w2t[...],
                       preferred_element_type=jnp.float32) + b2[...])
    y1_ref[...] = dv * jnp.dot(h, wg1t[...],
                               preferred_element_type=jnp.float32)
    dinv_ref[...] = dv


def _stage_c_body(y1_ref, sa, sb, dinv_ref, gb, wg2t, y2_ref):
    dv = dinv_ref[...]
    stot = sa[...] + sb[...] + y1_ref[...]
    h1 = _lrelu(dv * stot + gb[...])
    y2_ref[...] = dv * jnp.dot(h1, wg2t[...],
                               preferred_element_type=jnp.float32)


def _stage_e_body(y2_ref, sa, sb, dinv_ref, gb, wpt, bp, wat, ba,
                  h2_ref, out_ref, an_ref):
    dv = dinv_ref[...]
    stot = sa[...] + sb[...] + y2_ref[...]
    h2 = _lrelu(dv * stot + gb[...])
    h2_ref[...] = h2
    wsum = jnp.sum(wpt[...], axis=1, keepdims=True)
    out_ref[...] = jnp.dot(h2, wsum,
                           preferred_element_type=jnp.float32) + jnp.sum(bp[...])
    an_ref[...] = jnp.dot(h2, wat[...],
                          preferred_element_type=jnp.float32) + ba[...]


def _row_spec(w):
    return pl.BlockSpec((BLK, w), lambda i: (i, 0))


def _full_spec(shape):
    return pl.BlockSpec(shape, lambda i: tuple(0 for _ in shape))


def kernel(x, edge_index, W1, b1, W2, b2,
           gru1_Wih, gru1_Whh, gru1_bih, gru1_bhh, wt1_W, wt1_b, gcn1_b,
           gru2_Wih, gru2_Whh, gru2_bih, gru2_bhh, wt2_W, wt2_b, gcn2_b,
           Wp, bp, Wa, ba):
    f32 = jnp.float32

    src = edge_index[0]
    dst = edge_index[1]
    npad = E_PAD - E
    pad_idx = (N + jnp.arange(npad, dtype=src.dtype) % (N_PAD - N))
    src_p = jnp.concatenate([src, pad_idx]).reshape(NW * CPT, CHUNK)
    dst_p = jnp.concatenate([dst, pad_idx]).reshape(NW * CPT, CHUNK)

    x_p = jnp.pad(x, ((0, N_PAD - N), (0, 0)))
    zeros1 = jnp.zeros((N_PAD,), f32)
    zrows = jnp.zeros((RPT, H), f32)

    deg2 = _deg_kernel(dst_p, zeros1)
    degT = deg2.T

    def split3(b):
        return (b[0:16].reshape(1, 16), b[16:32].reshape(1, 16),
                b[32:48].reshape(1, 16))

    b1r, b1z, b1n = split3(gru1_bih)
    h1r, h1z, h1n = split3(gru1_bhh)
    b2r, b2z, b2n = split3(gru2_bih)
    h2r, h2z, h2n = split3(gru2_bhh)

    wgen = pl.pallas_call(
        _wgen_body,
        out_shape=jax.ShapeDtypeStruct((1, H * H), f32),
    )
    w1_flat = wgen(b1r, b1z, b1n, h1r, h1z, h1n, wt1_W.T, wt1_b.reshape(1, -1))
    w2_flat = wgen(b2r, b2z, b2n, h2r, h2z, h2n, wt2_W.T, wt2_b.reshape(1, -1))
    wg1t = w1_flat.reshape(H, H).T
    wg2t = w2_flat.reshape(H, H).T

    y1, dinv = pl.pallas_call(
        _stage_a_body,
        grid=(GRID,),
        in_specs=[
            _row_spec(D),
            _full_spec((D, 256)), _full_spec((1, 256)),
            _full_spec((256, H)), _full_spec((1, H)),
            _full_spec((H, H)),
            pl.BlockSpec((BLK, NC), lambda i: (i, 0)),
        ],
        out_specs=[_row_spec(H), pl.BlockSpec((BLK, 1), lambda i: (i, 0))],
        out_shape=[jax.ShapeDtypeStruct((N_PAD, H), f32),
                   jax.ShapeDtypeStruct((N_PAD, 1), f32)],
    )(x_p, W1.T, b1.reshape(1, -1), W2.T, b2.reshape(1, -1), wg1t, degT)

    s1 = _agg_kernel(y1, src_p, dst_p, zrows)

    y2 = pl.pallas_call(
        _stage_c_body,
        grid=(GRID,),
        in_specs=[
            _row_spec(H), _row_spec(H), _row_spec(H),
            pl.BlockSpec((BLK, 1), lambda i: (i, 0)),
            _full_spec((1, H)), _full_spec((H, H)),
        ],
        out_specs=_row_spec(H),
        out_shape=jax.ShapeDtypeStruct((N_PAD, H), f32),
    )(y1, s1[0], s1[1], dinv, gcn1_b.reshape(1, -1), wg2t)

    s2 = _agg_kernel(y2, src_p, dst_p, zrows)

    h2, out_c, an_c = pl.pallas_call(
        _stage_e_body,
        grid=(GRID,),
        in_specs=[
            _row_spec(H), _row_spec(H), _row_spec(H),
            pl.BlockSpec((BLK, 1), lambda i: (i, 0)),
            _full_spec((1, H)),
            _full_spec((H, 2)), _full_spec((1, 2)),
            _full_spec((H, 1)), _full_spec((1, 1)),
        ],
        out_specs=[_row_spec(H),
                   pl.BlockSpec((BLK, 1), lambda i: (i, 0)),
                   pl.BlockSpec((BLK, 1), lambda i: (i, 0))],
        out_shape=[jax.ShapeDtypeStruct((N_PAD, H), f32),
                   jax.ShapeDtypeStruct((N_PAD, 1), f32),
                   jax.ShapeDtypeStruct((N_PAD, 1), f32)],
    )(y2, s2[0], s2[1], dinv, gcn2_b.reshape(1, -1),
      Wp.T, bp.reshape(1, -1), Wa.T, ba.reshape(1, 1))

    return (out_c[:N, 0], an_c[:N, 0], h2[:N])
